# baseline (speedup 1.0000x reference)
import jax
import jax.numpy as jnp
from jax.experimental import pallas as pl
from jax.experimental.pallas import tpu as pltpu
from jax.experimental.pallas import tpu_sc as plsc

B = 1024
EMBED = 1024
EXDIM = 256
NCLS = 1000
NSUPPORT = 100000
K = 128

NS_PAD = 102400
S_BLK = 4096
B_BLK2 = 256
CH = 512
NCHUNK_BLK = S_BLK // CH
CMW = 128
CAP = 512
CHUNK = 12800
NW = 32
ROWS_W = B // NW


def _cmin_body(d2_ref, cmin_ref):
    d2v = d2_ref[...]
    mins = [jnp.min(d2v[:, c * CH:(c + 1) * CH], axis=1, keepdims=True)
            for c in range(NCHUNK_BLK)]
    pad = jnp.full((d2v.shape[0], CMW - NCHUNK_BLK), jnp.inf, jnp.float32)
    cmin_ref[...] = jnp.concatenate(mins + [pad], axis=1)


def _sc_filter(d2r_hbm, t0s_hbm, cval_hbm, cidx_hbm, t0_v, buf_v, buf2_v, cval_v, cidx_v, sem0, sem1):
    c = jax.lax.axis_index("c")
    s = jax.lax.axis_index("s")
    wid = s * 2 + c
    base = wid * ROWS_W
    pltpu.sync_copy(t0s_hbm.at[pl.ds(base * 16, ROWS_W * 16)], t0_v)

    def row_body(r, _):
        row = base + r

        def init_body(jj, _):
            cval_v[pl.ds(jj * 16, 16)] = jnp.full((16,), jnp.inf, jnp.float32)
            cidx_v[pl.ds(jj * 16, 16)] = jnp.full((16,), 2 ** 30, jnp.int32)
            return 0

        jax.lax.fori_loop(0, CAP // 16, init_body, 0, unroll=True)
        thr = t0_v[pl.ds(r * 16, 16)]

        bufs = (buf_v, buf2_v)
        sems = (sem0, sem1)
        nch = NS_PAD // CHUNK
        handles = [None, None]
        handles[0] = pltpu.async_copy(
            d2r_hbm.at[row * nch], bufs[0], sems[0])

        cnt = jnp.int32(0)
        for g in range(nch):
            handles[g % 2].wait()
            if g + 1 < nch:
                handles[(g + 1) % 2] = pltpu.async_copy(
                    d2r_hbm.at[row * nch + (g + 1)],
                    bufs[(g + 1) % 2], sems[(g + 1) % 2])
            cbuf = bufs[g % 2]

            def grp_body(i, cnt, g=g, cbuf=cbuf):
                vs = [cbuf[pl.ds(i * 128 + rr * 16, 16)] for rr in range(8)]
                ms = [v <= thr for v in vs]
                anym = ms[0]
                for rr in range(1, 8):
                    anym = anym | ms[rr]
                nany = plsc.all_reduce_population_count(anym)[0]

                def slow(cnt):
                    for rr in range(8):
                        cr = plsc.all_reduce_population_count(ms[rr])[0]
                        iv = jax.lax.iota(jnp.int32, 16) + (
                            g * CHUNK + i * 128 + rr * 16)
                        off = jnp.minimum(cnt, CAP - 16)
                        plsc.store_compressed(
                            cval_v.at[pl.ds(off, 16)], vs[rr], mask=ms[rr])
                        plsc.store_compressed(
                            cidx_v.at[pl.ds(off, 16)], iv, mask=ms[rr])
                        cnt = cnt + cr
                    return cnt

                return jax.lax.cond(nany > 0, slow, lambda cc: cc, cnt)

            cnt = jax.lax.fori_loop(0, CHUNK // 128, grp_body, cnt)
        pltpu.sync_copy(cval_v, cval_hbm.at[row])
        pltpu.sync_copy(cidx_v, cidx_hbm.at[row])
        return 0

    jax.lax.fori_loop(0, ROWS_W, row_body, 0)


def kernel(x, support_exemplar_vectors, conv_weight, conv_bias, fc_weight, fc_bias, k):
    del k
    sup = support_exemplar_vectors
    w2d = conv_weight[:, 0, :]
    ex = x @ w2d.T + conv_bias
    logits = ex @ fc_weight.T + fc_bias
    q_sq = jnp.sum(ex * ex, axis=1, keepdims=True)
    s_sq = jnp.sum(sup * sup, axis=1)
    d2 = q_sq + s_sq[None, :] - 2.0 * (ex @ sup.T)
    d2p = jnp.pad(d2, ((0, 0), (0, NS_PAD - NSUPPORT)),
                  constant_values=jnp.inf)

    cmin = pl.pallas_call(
        _cmin_body,
        grid=(B // B_BLK2, NS_PAD // S_BLK),
        in_specs=[pl.BlockSpec((B_BLK2, S_BLK), lambda b, j: (b, j))],
        out_specs=pl.BlockSpec((B_BLK2, CMW), lambda b, j: (b, j)),
        out_shape=jax.ShapeDtypeStruct((B, (NS_PAD // S_BLK) * CMW),
                                       jnp.float32),
        compiler_params=pltpu.CompilerParams(
            dimension_semantics=("parallel", "arbitrary")),
    )(d2p)

    t0 = -jax.lax.top_k(-cmin, K)[0][:, K - 1]
    t0s = jnp.broadcast_to(t0[:, None], (B, 16)).reshape(B * 16)

    d2r = d2p.reshape(B * (NS_PAD // CHUNK), CHUNK)
    mesh = plsc.VectorSubcoreMesh(core_axis_name="c", subcore_axis_name="s")
    cval, cidx = pl.kernel(
        _sc_filter,
        out_type=[
            jax.ShapeDtypeStruct((B, CAP), jnp.float32),
            jax.ShapeDtypeStruct((B, CAP), jnp.int32),
        ],
        mesh=mesh,
        scratch_types=[
            pltpu.VMEM((ROWS_W * 16,), jnp.float32),
            pltpu.VMEM((CHUNK,), jnp.float32),
            pltpu.VMEM((CHUNK,), jnp.float32),
            pltpu.VMEM((CAP,), jnp.float32),
            pltpu.VMEM((CAP,), jnp.int32),
            pltpu.SemaphoreType.DMA,
            pltpu.SemaphoreType.DMA,
        ],
        compiler_params=pltpu.CompilerParams(needs_layout_passes=False),
    )(d2r, t0s)

    neg_vals, pos = jax.lax.top_k(-cval, K)
    top_vals = -neg_vals
    top_idx = jnp.take_along_axis(cidx, pos, axis=1)
    return logits, top_vals, top_idx


# speedup vs baseline: 1.0523x; 1.0523x over previous
import jax
import jax.numpy as jnp
from jax.experimental import pallas as pl
from jax.experimental.pallas import tpu as pltpu
from jax.experimental.pallas import tpu_sc as plsc

B = 1024
EMBED = 1024
EXDIM = 256
NCLS = 1000
NSUPPORT = 100000
K = 128

NS_PAD = 102400
S_BLK = 4096
B_BLK2 = 256
CH = 512
NCHUNK_BLK = S_BLK // CH
CMW = 128
CAP = 512
CHUNK = 12800
NW = 32
ROWS_W = B // NW


def _cmin_body(d2_ref, d2p_ref, cmin_ref):
    j = pl.program_id(1)
    raw = d2_ref[...]
    col = j * S_BLK + jax.lax.broadcasted_iota(jnp.int32, raw.shape, 1)
    d2v = jnp.where(col < NSUPPORT, raw, jnp.inf)
    d2p_ref[...] = d2v
    mins = [jnp.min(d2v[:, c * CH:(c + 1) * CH], axis=1, keepdims=True)
            for c in range(NCHUNK_BLK)]
    pad = jnp.full((d2v.shape[0], CMW - NCHUNK_BLK), jnp.inf, jnp.float32)
    cmin_ref[...] = jnp.concatenate(mins + [pad], axis=1)


def _sc_filter(d2r_hbm, t0s_hbm, cval_hbm, cidx_hbm, t0_v, buf_v, buf2_v, cval_v, cidx_v, sem0, sem1):
    c = jax.lax.axis_index("c")
    s = jax.lax.axis_index("s")
    wid = s * 2 + c
    base = wid * ROWS_W
    pltpu.sync_copy(t0s_hbm.at[pl.ds(base * 16, ROWS_W * 16)], t0_v)

    def row_body(r, _):
        row = base + r

        def init_body(jj, _):
            cval_v[pl.ds(jj * 16, 16)] = jnp.full((16,), jnp.inf, jnp.float32)
            cidx_v[pl.ds(jj * 16, 16)] = jnp.full((16,), 2 ** 30, jnp.int32)
            return 0

        jax.lax.fori_loop(0, CAP // 16, init_body, 0, unroll=True)
        thr = t0_v[pl.ds(r * 16, 16)]

        bufs = (buf_v, buf2_v)
        sems = (sem0, sem1)
        nch = NS_PAD // CHUNK
        handles = [None, None]
        handles[0] = pltpu.async_copy(
            d2r_hbm.at[row * nch], bufs[0], sems[0])

        cnt = jnp.int32(0)
        for g in range(nch):
            handles[g % 2].wait()
            if g + 1 < nch:
                handles[(g + 1) % 2] = pltpu.async_copy(
                    d2r_hbm.at[row * nch + (g + 1)],
                    bufs[(g + 1) % 2], sems[(g + 1) % 2])
            cbuf = bufs[g % 2]

            def grp_body(i, cnt, g=g, cbuf=cbuf):
                vs = [cbuf[pl.ds(i * 128 + rr * 16, 16)] for rr in range(8)]
                ms = [v <= thr for v in vs]
                anym = ms[0]
                for rr in range(1, 8):
                    anym = anym | ms[rr]
                nany = plsc.all_reduce_population_count(anym)[0]

                def slow(cnt):
                    for rr in range(8):
                        cr = plsc.all_reduce_population_count(ms[rr])[0]
                        iv = jax.lax.iota(jnp.int32, 16) + (
                            g * CHUNK + i * 128 + rr * 16)
                        off = jnp.minimum(cnt, CAP - 16)
                        plsc.store_compressed(
                            cval_v.at[pl.ds(off, 16)], vs[rr], mask=ms[rr])
                        plsc.store_compressed(
                            cidx_v.at[pl.ds(off, 16)], iv, mask=ms[rr])
                        cnt = cnt + cr
                    return cnt

                return jax.lax.cond(nany > 0, slow, lambda cc: cc, cnt)

            cnt = jax.lax.fori_loop(0, CHUNK // 128, grp_body, cnt)
        pltpu.sync_copy(cval_v, cval_hbm.at[row])
        pltpu.sync_copy(cidx_v, cidx_hbm.at[row])
        return 0

    jax.lax.fori_loop(0, ROWS_W, row_body, 0)


def kernel(x, support_exemplar_vectors, conv_weight, conv_bias, fc_weight, fc_bias, k):
    del k
    sup = support_exemplar_vectors
    w2d = conv_weight[:, 0, :]
    ex = x @ w2d.T + conv_bias
    logits = ex @ fc_weight.T + fc_bias
    q_sq = jnp.sum(ex * ex, axis=1, keepdims=True)
    s_sq = jnp.sum(sup * sup, axis=1)
    d2 = q_sq + s_sq[None, :] - 2.0 * (ex @ sup.T)

    d2p, cmin = pl.pallas_call(
        _cmin_body,
        grid=(B // B_BLK2, NS_PAD // S_BLK),
        in_specs=[pl.BlockSpec((B_BLK2, S_BLK), lambda b, j: (b, j))],
        out_specs=[
            pl.BlockSpec((B_BLK2, S_BLK), lambda b, j: (b, j)),
            pl.BlockSpec((B_BLK2, CMW), lambda b, j: (b, j)),
        ],
        out_shape=[
            jax.ShapeDtypeStruct((B, NS_PAD), jnp.float32),
            jax.ShapeDtypeStruct((B, (NS_PAD // S_BLK) * CMW), jnp.float32),
        ],
        compiler_params=pltpu.CompilerParams(
            dimension_semantics=("parallel", "arbitrary")),
    )(d2)

    t0 = -jax.lax.top_k(-cmin, K)[0][:, K - 1]
    t0s = jnp.broadcast_to(t0[:, None], (B, 16)).reshape(B * 16)

    d2r = d2p.reshape(B * (NS_PAD // CHUNK), CHUNK)
    mesh = plsc.VectorSubcoreMesh(core_axis_name="c", subcore_axis_name="s")
    cval, cidx = pl.kernel(
        _sc_filter,
        out_type=[
            jax.ShapeDtypeStruct((B, CAP), jnp.float32),
            jax.ShapeDtypeStruct((B, CAP), jnp.int32),
        ],
        mesh=mesh,
        scratch_types=[
            pltpu.VMEM((ROWS_W * 16,), jnp.float32),
            pltpu.VMEM((CHUNK,), jnp.float32),
            pltpu.VMEM((CHUNK,), jnp.float32),
            pltpu.VMEM((CAP,), jnp.float32),
            pltpu.VMEM((CAP,), jnp.int32),
            pltpu.SemaphoreType.DMA,
            pltpu.SemaphoreType.DMA,
        ],
        compiler_params=pltpu.CompilerParams(needs_layout_passes=False),
    )(d2r, t0s)

    neg_vals, pos = jax.lax.top_k(-cval, K)
    top_vals = -neg_vals
    top_idx = jnp.take_along_axis(cidx, pos, axis=1)
    return logits, top_vals, top_idx
